# baseline (speedup 1.0000x reference)
import functools

import jax
import jax.numpy as jnp
from jax import lax
from jax.experimental import pallas as pl
from jax.experimental.pallas import tpu as pltpu
from jax.experimental.pallas import tpu_sc as plsc

N = 10000
E = 320000
D = 128
C = 64
CH = C // 2
NC = 2
NS = 16
NW = NC * NS
CHUNK = 128
NCH = 81
NCH2 = 162
EPAD = NW * NCH * CHUNK
NPAD = 10112
RPT = NPAD // NS

_MESH = plsc.VectorSubcoreMesh(
    core_axis_name="c", subcore_axis_name="s", num_cores=NC, num_subcores=NS
)
_SC_PARAMS = pltpu.CompilerParams(use_tc_tiling_on_sc=False,
                                  needs_layout_passes=False)


def _zero_fill(buf, rows, width):
    zeros16 = jnp.zeros((16,), jnp.float32)

    @pl.loop(0, rows)
    def _(r):
        @pl.loop(0, width // 16)
        def _(q):
            buf[r, pl.ds(q * 16, 16)] = zeros16


def _init_acc_rows(rows_v, acc_sh, base):
    nfull = RPT // CHUNK
    rem = RPT - nfull * CHUNK
    for k in range(nfull):
        pltpu.sync_copy(rows_v, acc_sh.at[pl.ds(base + k * CHUNK, CHUNK)])
    pltpu.sync_copy(rows_v.at[pl.ds(0, rem)],
                    acc_sh.at[pl.ds(base + nfull * CHUNK, rem)])


def _ring(g_sh, acc_sh, src_v, dst_v, rows_v, semg, sems):
    pltpu.async_copy(g_sh.at[src_v.at[0]], rows_v.at[0], semg)
    pltpu.async_copy(g_sh.at[src_v.at[1]], rows_v.at[1], semg)

    @pl.loop(0, NCH2 // 3)
    def _(p):
        j0 = 3 * p
        for b in range(3):
            jj = j0 + b
            buf = rows_v.at[b]
            pltpu.make_async_copy(g_sh.at[src_v.at[jj]], buf, semg).wait()

            @pl.when(jj >= 1)
            def _():
                pltpu.make_async_copy(rows_v.at[(b + 2) % 3],
                                      acc_sh.at[dst_v.at[jj]], sems).wait()

            pltpu.async_copy(buf, acc_sh.at[dst_v.at[jj]], sems, add=True)

            @pl.when(jj + 2 < NCH2)
            def _():
                pltpu.async_copy(g_sh.at[src_v.at[jj + 2]],
                                 rows_v.at[(b + 2) % 3], semg)

    pltpu.make_async_copy(rows_v.at[(NCH2 - 1) % 3],
                          acc_sh.at[dst_v.at[0]], sems).wait()
    plsc.subcore_barrier()


def _splat(vec_ref, r):
    idx = jnp.zeros((16,), jnp.int32) + r
    return plsc.load_gather(vec_ref, [idx])


@functools.partial(
    pl.kernel,
    out_type=jax.ShapeDtypeStruct((NC, NPAD, CH), jnp.float32),
    mesh=_MESH,
    scratch_types=[
        pltpu.VMEM((NCH2, CHUNK), jnp.int32),
        pltpu.VMEM((NCH2, CHUNK), jnp.int32),
        pltpu.VMEM((3, CHUNK, CH), jnp.float32),
        pltpu.VMEM((CHUNK, CH), jnp.float32),
        pltpu.VMEM((CHUNK, 16), jnp.float32),
        pltpu.VMEM((CHUNK, 16), jnp.float32),
        pltpu.VMEM((RPT, ), jnp.float32),
        pltpu.VMEM((RPT, ), jnp.float32),
        pltpu.VMEM_SHARED((NPAD, CH), jnp.float32),
        pltpu.VMEM_SHARED((NPAD, CH), jnp.float32),
        pltpu.VMEM_SHARED((NPAD, 16), jnp.float32),
        pltpu.SemaphoreType.DMA,
        pltpu.SemaphoreType.DMA,
        pltpu.SemaphoreType.DMA,
    ],
    compiler_params=_SC_PARAMS,
)
def _hops(z_hbm, srcr_hbm, dstr_hbm, out_hbm,
          src_v, dst_v, rows_v, ta, th0, th1, tsv, tiv, acc_sh, g_sh,
          hist_sh, semi, semg, sems):
    core = lax.axis_index("c")
    sid = lax.axis_index("s")
    base = sid * RPT
    pltpu.async_copy(srcr_hbm.at[sid], src_v, semi)
    pltpu.async_copy(dstr_hbm.at[sid], dst_v, semi)

    nfull = RPT // CHUNK
    rem = RPT - nfull * CHUNK
    zeros16i = jnp.zeros((16,), jnp.int32)
    iota16 = lax.iota(jnp.int32, 16)

    _zero_fill(th0, CHUNK, 16)
    _init_acc_rows(th0, hist_sh, base)
    ones16 = jnp.ones((16,), jnp.float32)

    @pl.loop(0, CHUNK)
    def _(r):
        th1[r, pl.ds(0, 16)] = ones16

    pltpu.make_async_copy(dstr_hbm.at[sid], dst_v, semi).wait()
    plsc.subcore_barrier()

    @pl.loop(0, NCH2)
    def _(j):
        pltpu.sync_copy(th1, hist_sh.at[dst_v.at[j]], add=True)

    plsc.subcore_barrier()

    for k in range(nfull + 1):
        nb = CHUNK if k < nfull else rem
        blk = pl.ds(base + k * CHUNK, nb)
        tab = ta.at[pl.ds(0, nb)]
        pltpu.sync_copy(z_hbm.at[core].at[blk], tab)
        pltpu.sync_copy(hist_sh.at[blk], th0.at[pl.ds(0, nb)])

        ngf = nb // 16
        ntail = nb - ngf * 16

        def _group(g16, mask):
            ridx = g16 * 16 + iota16
            cnt = plsc.load_gather(th0, [ridx, zeros16i], mask=mask)
            deg = cnt + 1.0
            inv = 1.0 / deg
            ibits = plsc.bitcast(deg, jnp.int32)
            y = plsc.bitcast(0x5F3759DF - lax.shift_right_logical(ibits, 1),
                             jnp.float32)
            for _ in range(3):
                y = y * (1.5 - 0.5 * deg * y * y)
            sk = k * CHUNK + g16 * 16
            plsc.store_scatter(tsv, [sk + iota16], y, mask=mask)
            plsc.store_scatter(tiv, [sk + iota16], inv, mask=mask)

        @pl.loop(0, ngf)
        def _(g16):
            _group(g16, iota16 >= 0)

        if ntail:
            _group(ngf, iota16 < ntail)

        @pl.loop(0, nb)
        def _(r):
            sv = _splat(tsv, k * CHUNK + r)
            for q in range(CH // 16):
                sl = pl.ds(q * 16, 16)
                ta[r, sl] = ta[r, sl] * sv

        pltpu.sync_copy(tab, g_sh.at[blk])
        pltpu.sync_copy(tab, acc_sh.at[blk])

    pltpu.make_async_copy(srcr_hbm.at[sid], src_v, semi).wait()
    plsc.subcore_barrier()

    _ring(g_sh, acc_sh, src_v, dst_v, rows_v, semg, sems)

    for k in range(nfull + 1):
        nb = CHUNK if k < nfull else rem
        blk = pl.ds(base + k * CHUNK, nb)
        tab = ta.at[pl.ds(0, nb)]
        pltpu.sync_copy(acc_sh.at[blk], tab)

        @pl.loop(0, nb)
        def _(r):
            iv = _splat(tiv, k * CHUNK + r)
            for q in range(CH // 16):
                sl = pl.ds(q * 16, 16)
                ta[r, sl] = ta[r, sl] * iv

        pltpu.sync_copy(tab, g_sh.at[blk])
        pltpu.sync_copy(tab, acc_sh.at[blk])

    plsc.subcore_barrier()

    _ring(g_sh, acc_sh, src_v, dst_v, rows_v, semg, sems)

    for k in range(nfull + 1):
        nb = CHUNK if k < nfull else rem
        blk = pl.ds(base + k * CHUNK, nb)
        tab = ta.at[pl.ds(0, nb)]
        pltpu.sync_copy(acc_sh.at[blk], tab)

        @pl.loop(0, nb)
        def _(r):
            sv = _splat(tsv, k * CHUNK + r)
            for q in range(CH // 16):
                sl = pl.ds(q * 16, 16)
                ta[r, sl] = ta[r, sl] * sv

        pltpu.sync_copy(tab, out_hbm.at[core].at[blk])


def _mm_body(x_ref, wt_ref, z_ref):
    z = lax.dot_general(
        x_ref[...], wt_ref[...], (((1,), (0,)), ((), ())),
        precision=lax.Precision.HIGHEST,
        preferred_element_type=jnp.float32)
    z_ref[0] = z[:, :CH]
    z_ref[1] = z[:, CH:]


def _final_body(h_ref, b_ref, out_ref):
    h = jnp.concatenate([h_ref[0], h_ref[1]], axis=1)
    logits = h + b_ref[...]
    m = jnp.max(logits, axis=1, keepdims=True)
    shifted = logits - m
    lse = jnp.log(jnp.sum(jnp.exp(shifted), axis=1, keepdims=True))
    out_ref[...] = shifted - lse


def kernel(x, edge_index, W, b):
    src = edge_index[0]
    dst = edge_index[1]
    pad = EPAD - E
    srcp = jnp.concatenate([src, jnp.zeros((pad,), jnp.int32)])
    dstp = jnp.concatenate([dst, jnp.full((pad,), N, jnp.int32)])
    srcr16 = srcp.reshape(NS, NCH2, CHUNK)
    dstr16 = dstp.reshape(NS, NCH2, CHUNK)

    xpad = jnp.concatenate([x, jnp.zeros((NPAD - N, D), jnp.float32)])
    RB = NPAD // 8
    zc = pl.pallas_call(
        _mm_body,
        grid=(8,),
        in_specs=[
            pl.BlockSpec((RB, D), lambda i: (i, 0)),
            pl.BlockSpec((D, C), lambda i: (0, 0)),
        ],
        out_specs=pl.BlockSpec((NC, RB, CH), lambda i: (0, i, 0)),
        out_shape=jax.ShapeDtypeStruct((NC, NPAD, CH), jnp.float32),
    )(xpad, W.T)

    h = _hops(zc, srcr16, dstr16)

    FB = N // 10
    out = pl.pallas_call(
        _final_body,
        grid=(10,),
        in_specs=[
            pl.BlockSpec((NC, FB, CH), lambda i: (0, i, 0)),
            pl.BlockSpec((1, C), lambda i: (0, 0)),
        ],
        out_specs=pl.BlockSpec((FB, C), lambda i: (i, 0)),
        out_shape=jax.ShapeDtypeStruct((N, C), jnp.float32),
    )(h, b.reshape(1, C))

    return out


# speedup vs baseline: 1.0011x; 1.0011x over previous
import functools

import jax
import jax.numpy as jnp
from jax import lax
from jax.experimental import pallas as pl
from jax.experimental.pallas import tpu as pltpu
from jax.experimental.pallas import tpu_sc as plsc

N = 10000
E = 320000
D = 128
C = 64
CH = C // 2
NC = 2
NS = 16
NW = NC * NS
CHUNK = 128
NCH = 81
NCH2 = 162
EPAD = NW * NCH * CHUNK
NPAD = 10112
RPT = NPAD // NS

_MESH = plsc.VectorSubcoreMesh(
    core_axis_name="c", subcore_axis_name="s", num_cores=NC, num_subcores=NS
)
_SC_PARAMS = pltpu.CompilerParams(use_tc_tiling_on_sc=False,
                                  needs_layout_passes=False)


def _zero_fill(buf, rows, width):
    zeros16 = jnp.zeros((16,), jnp.float32)

    @pl.loop(0, rows)
    def _(r):
        @pl.loop(0, width // 16)
        def _(q):
            buf[r, pl.ds(q * 16, 16)] = zeros16


def _init_acc_rows(rows_v, acc_sh, base):
    nfull = RPT // CHUNK
    rem = RPT - nfull * CHUNK
    for k in range(nfull):
        pltpu.sync_copy(rows_v, acc_sh.at[pl.ds(base + k * CHUNK, CHUNK)])
    pltpu.sync_copy(rows_v.at[pl.ds(0, rem)],
                    acc_sh.at[pl.ds(base + nfull * CHUNK, rem)])


def _ring(g_sh, acc_sh, src_v, dst_v, rows_v, semg, sems):
    pltpu.async_copy(g_sh.at[src_v.at[0]], rows_v.at[0], semg)
    pltpu.async_copy(g_sh.at[src_v.at[1]], rows_v.at[1], semg)

    @pl.loop(0, NCH2 // 3)
    def _(p):
        j0 = 3 * p
        for b in range(3):
            jj = j0 + b
            buf = rows_v.at[b]
            pltpu.make_async_copy(g_sh.at[src_v.at[jj]], buf, semg).wait()

            @pl.when(jj >= 1)
            def _():
                pltpu.make_async_copy(rows_v.at[(b + 2) % 3],
                                      acc_sh.at[dst_v.at[jj]], sems).wait()

            pltpu.async_copy(buf, acc_sh.at[dst_v.at[jj]], sems, add=True)

            @pl.when(jj + 2 < NCH2)
            def _():
                pltpu.async_copy(g_sh.at[src_v.at[jj + 2]],
                                 rows_v.at[(b + 2) % 3], semg)

    pltpu.make_async_copy(rows_v.at[(NCH2 - 1) % 3],
                          acc_sh.at[dst_v.at[0]], sems).wait()
    plsc.subcore_barrier()


def _splat(vec_ref, r):
    idx = jnp.zeros((16,), jnp.int32) + r
    return plsc.load_gather(vec_ref, [idx])


@functools.partial(
    pl.kernel,
    out_type=jax.ShapeDtypeStruct((NC, NPAD, CH), jnp.float32),
    mesh=_MESH,
    scratch_types=[
        pltpu.VMEM((NCH2, CHUNK), jnp.int32),
        pltpu.VMEM((NCH2, CHUNK), jnp.int32),
        pltpu.VMEM((3, CHUNK, CH), jnp.float32),
        pltpu.VMEM((CHUNK, CH), jnp.float32),
        pltpu.VMEM((CHUNK, 16), jnp.float32),
        pltpu.VMEM((CHUNK, 16), jnp.float32),
        pltpu.VMEM((RPT, ), jnp.float32),
        pltpu.VMEM((RPT, ), jnp.float32),
        pltpu.VMEM_SHARED((NPAD, CH), jnp.float32),
        pltpu.VMEM_SHARED((NPAD, CH), jnp.float32),
        pltpu.VMEM_SHARED((NPAD, 16), jnp.float32),
        pltpu.SemaphoreType.DMA,
        pltpu.SemaphoreType.DMA,
        pltpu.SemaphoreType.DMA,
    ],
    compiler_params=_SC_PARAMS,
)
def _hops(z_hbm, srcr_hbm, dstr_hbm, out_hbm,
          src_v, dst_v, rows_v, ta, th0, th1, tsv, tiv, acc_sh, g_sh,
          hist_sh, semi, semg, sems):
    core = lax.axis_index("c")
    sid = lax.axis_index("s")
    base = sid * RPT
    pltpu.async_copy(srcr_hbm.at[sid], src_v, semi)
    pltpu.async_copy(dstr_hbm.at[sid], dst_v, semi)

    nfull = RPT // CHUNK
    rem = RPT - nfull * CHUNK
    zeros16i = jnp.zeros((16,), jnp.int32)
    iota16 = lax.iota(jnp.int32, 16)

    _zero_fill(th0, CHUNK, 16)
    _init_acc_rows(th0, hist_sh, base)
    ones16 = jnp.ones((16,), jnp.float32)

    @pl.loop(0, CHUNK)
    def _(r):
        th1[r, pl.ds(0, 16)] = ones16

    pltpu.make_async_copy(dstr_hbm.at[sid], dst_v, semi).wait()
    plsc.subcore_barrier()

    @pl.loop(0, NCH2 // 6)
    def _(p):
        j0 = 6 * p
        for b in range(6):
            pltpu.async_copy(th1, hist_sh.at[dst_v.at[j0 + b]], sems,
                             add=True)
        for b in range(6):
            pltpu.make_async_copy(th1, hist_sh.at[dst_v.at[j0 + b]],
                                  sems).wait()

    plsc.subcore_barrier()

    for k in range(nfull + 1):
        nb = CHUNK if k < nfull else rem
        blk = pl.ds(base + k * CHUNK, nb)
        tab = ta.at[pl.ds(0, nb)]
        pltpu.sync_copy(z_hbm.at[core].at[blk], tab)
        pltpu.sync_copy(hist_sh.at[blk], th0.at[pl.ds(0, nb)])

        ngf = nb // 16
        ntail = nb - ngf * 16

        def _group(g16, mask):
            ridx = g16 * 16 + iota16
            cnt = plsc.load_gather(th0, [ridx, zeros16i], mask=mask)
            deg = cnt + 1.0
            inv = 1.0 / deg
            ibits = plsc.bitcast(deg, jnp.int32)
            y = plsc.bitcast(0x5F3759DF - lax.shift_right_logical(ibits, 1),
                             jnp.float32)
            for _ in range(3):
                y = y * (1.5 - 0.5 * deg * y * y)
            sk = k * CHUNK + g16 * 16
            plsc.store_scatter(tsv, [sk + iota16], y, mask=mask)
            plsc.store_scatter(tiv, [sk + iota16], inv, mask=mask)

        @pl.loop(0, ngf)
        def _(g16):
            _group(g16, iota16 >= 0)

        if ntail:
            _group(ngf, iota16 < ntail)

        @pl.loop(0, nb)
        def _(r):
            sv = _splat(tsv, k * CHUNK + r)
            for q in range(CH // 16):
                sl = pl.ds(q * 16, 16)
                ta[r, sl] = ta[r, sl] * sv

        pltpu.sync_copy(tab, g_sh.at[blk])
        pltpu.sync_copy(tab, acc_sh.at[blk])

    pltpu.make_async_copy(srcr_hbm.at[sid], src_v, semi).wait()
    plsc.subcore_barrier()

    _ring(g_sh, acc_sh, src_v, dst_v, rows_v, semg, sems)

    for k in range(nfull + 1):
        nb = CHUNK if k < nfull else rem
        blk = pl.ds(base + k * CHUNK, nb)
        tab = ta.at[pl.ds(0, nb)]
        pltpu.sync_copy(acc_sh.at[blk], tab)

        @pl.loop(0, nb)
        def _(r):
            iv = _splat(tiv, k * CHUNK + r)
            for q in range(CH // 16):
                sl = pl.ds(q * 16, 16)
                ta[r, sl] = ta[r, sl] * iv

        pltpu.sync_copy(tab, g_sh.at[blk])
        pltpu.sync_copy(tab, acc_sh.at[blk])

    plsc.subcore_barrier()

    _ring(g_sh, acc_sh, src_v, dst_v, rows_v, semg, sems)

    for k in range(nfull + 1):
        nb = CHUNK if k < nfull else rem
        blk = pl.ds(base + k * CHUNK, nb)
        tab = ta.at[pl.ds(0, nb)]
        pltpu.sync_copy(acc_sh.at[blk], tab)

        @pl.loop(0, nb)
        def _(r):
            sv = _splat(tsv, k * CHUNK + r)
            for q in range(CH // 16):
                sl = pl.ds(q * 16, 16)
                ta[r, sl] = ta[r, sl] * sv

        pltpu.sync_copy(tab, out_hbm.at[core].at[blk])


def _mm_body(x_ref, wt_ref, z_ref):
    z = lax.dot_general(
        x_ref[...], wt_ref[...], (((1,), (0,)), ((), ())),
        precision=lax.Precision.HIGHEST,
        preferred_element_type=jnp.float32)
    z_ref[0] = z[:, :CH]
    z_ref[1] = z[:, CH:]


def _final_body(h_ref, b_ref, out_ref):
    h = jnp.concatenate([h_ref[0], h_ref[1]], axis=1)
    logits = h + b_ref[...]
    m = jnp.max(logits, axis=1, keepdims=True)
    shifted = logits - m
    lse = jnp.log(jnp.sum(jnp.exp(shifted), axis=1, keepdims=True))
    out_ref[...] = shifted - lse


def kernel(x, edge_index, W, b):
    src = edge_index[0]
    dst = edge_index[1]
    pad = EPAD - E
    srcp = jnp.concatenate([src, jnp.zeros((pad,), jnp.int32)])
    dstp = jnp.concatenate([dst, jnp.full((pad,), N, jnp.int32)])
    srcr16 = srcp.reshape(NS, NCH2, CHUNK)
    dstr16 = dstp.reshape(NS, NCH2, CHUNK)

    xpad = jnp.concatenate([x, jnp.zeros((NPAD - N, D), jnp.float32)])
    RB = NPAD // 8
    zc = pl.pallas_call(
        _mm_body,
        grid=(8,),
        in_specs=[
            pl.BlockSpec((RB, D), lambda i: (i, 0)),
            pl.BlockSpec((D, C), lambda i: (0, 0)),
        ],
        out_specs=pl.BlockSpec((NC, RB, CH), lambda i: (0, i, 0)),
        out_shape=jax.ShapeDtypeStruct((NC, NPAD, CH), jnp.float32),
    )(xpad, W.T)

    h = _hops(zc, srcr16, dstr16)

    FB = N // 10
    out = pl.pallas_call(
        _final_body,
        grid=(10,),
        in_specs=[
            pl.BlockSpec((NC, FB, CH), lambda i: (0, i, 0)),
            pl.BlockSpec((1, C), lambda i: (0, 0)),
        ],
        out_specs=pl.BlockSpec((FB, C), lambda i: (i, 0)),
        out_shape=jax.ShapeDtypeStruct((N, C), jnp.float32),
    )(h, b.reshape(1, C))

    return out


# speedup vs baseline: 1.0715x; 1.0703x over previous
import functools

import jax
import jax.numpy as jnp
from jax import lax
from jax.experimental import pallas as pl
from jax.experimental.pallas import tpu as pltpu
from jax.experimental.pallas import tpu_sc as plsc

N = 10000
E = 320000
D = 128
C = 64
CH = C // 2
NC = 2
NS = 16
NW = NC * NS
CHUNK = 128
NCH = 81
NCH2 = 162
EPAD = NW * NCH * CHUNK
NPAD = 10112
RPT = NPAD // NS

_MESH = plsc.VectorSubcoreMesh(
    core_axis_name="c", subcore_axis_name="s", num_cores=NC, num_subcores=NS
)
_SC_PARAMS = pltpu.CompilerParams(use_tc_tiling_on_sc=False,
                                  needs_layout_passes=False)


def _zero_fill(buf, rows, width):
    zeros16 = jnp.zeros((16,), jnp.float32)

    @pl.loop(0, rows)
    def _(r):
        @pl.loop(0, width // 16)
        def _(q):
            buf[r, pl.ds(q * 16, 16)] = zeros16


def _init_acc_rows(rows_v, acc_sh, base):
    nfull = RPT // CHUNK
    rem = RPT - nfull * CHUNK
    for k in range(nfull):
        pltpu.sync_copy(rows_v, acc_sh.at[pl.ds(base + k * CHUNK, CHUNK)])
    pltpu.sync_copy(rows_v.at[pl.ds(0, rem)],
                    acc_sh.at[pl.ds(base + nfull * CHUNK, rem)])


@functools.partial(
    pl.kernel,
    out_type=jax.ShapeDtypeStruct((NC, NPAD, 16), jnp.float32),
    mesh=_MESH,
    scratch_types=[
        pltpu.VMEM((NCH, CHUNK), jnp.int32),
        pltpu.VMEM((CHUNK, 16), jnp.float32),
        pltpu.VMEM_SHARED((NPAD, 16), jnp.float32),
    ],
    compiler_params=_SC_PARAMS,
)
def _hist(dstr_hbm, out_hbm, dst_v, ones_v, acc_sh):
    core = lax.axis_index("c")
    sid = lax.axis_index("s")
    wid = core * NS + sid
    base = sid * RPT
    pltpu.sync_copy(dstr_hbm.at[wid], dst_v)
    _zero_fill(ones_v, CHUNK, 16)
    _init_acc_rows(ones_v, acc_sh, base)
    ones16 = jnp.ones((16,), jnp.float32)

    @pl.loop(0, CHUNK)
    def _(r):
        ones_v[r, pl.ds(0, 16)] = ones16

    plsc.subcore_barrier()

    @pl.loop(0, NCH)
    def _(j):
        pltpu.sync_copy(ones_v, acc_sh.at[dst_v.at[j]], add=True)

    plsc.subcore_barrier()
    pltpu.sync_copy(acc_sh.at[pl.ds(base, RPT)],
                    out_hbm.at[core].at[pl.ds(base, RPT)])


def _ring(g_sh, acc_sh, src_v, dst_v, rows_v, semg, sems):
    pltpu.async_copy(g_sh.at[src_v.at[0]], rows_v.at[0], semg)
    pltpu.async_copy(g_sh.at[src_v.at[1]], rows_v.at[1], semg)

    @pl.loop(0, NCH2 // 3)
    def _(p):
        j0 = 3 * p
        for b in range(3):
            jj = j0 + b
            buf = rows_v.at[b]
            pltpu.make_async_copy(g_sh.at[src_v.at[jj]], buf, semg).wait()

            @pl.when(jj >= 1)
            def _():
                pltpu.make_async_copy(rows_v.at[(b + 2) % 3],
                                      acc_sh.at[dst_v.at[jj]], sems).wait()

            pltpu.async_copy(buf, acc_sh.at[dst_v.at[jj]], sems, add=True)

            @pl.when(jj + 2 < NCH2)
            def _():
                pltpu.async_copy(g_sh.at[src_v.at[jj + 2]],
                                 rows_v.at[(b + 2) % 3], semg)

    pltpu.make_async_copy(rows_v.at[(NCH2 - 1) % 3],
                          acc_sh.at[dst_v.at[0]], sems).wait()
    plsc.subcore_barrier()


def _splat(vec_ref, r):
    idx = jnp.zeros((16,), jnp.int32) + r
    return plsc.load_gather(vec_ref, [idx])


@functools.partial(
    pl.kernel,
    out_type=jax.ShapeDtypeStruct((NC, NPAD, CH), jnp.float32),
    mesh=_MESH,
    scratch_types=[
        pltpu.VMEM((NCH2, CHUNK), jnp.int32),
        pltpu.VMEM((NCH2, CHUNK), jnp.int32),
        pltpu.VMEM((3, CHUNK, CH), jnp.float32),
        pltpu.VMEM((CHUNK, CH), jnp.float32),
        pltpu.VMEM((CHUNK, 16), jnp.float32),
        pltpu.VMEM((CHUNK, 16), jnp.float32),
        pltpu.VMEM((RPT, ), jnp.float32),
        pltpu.VMEM((RPT, ), jnp.float32),
        pltpu.VMEM_SHARED((NPAD, CH), jnp.float32),
        pltpu.VMEM_SHARED((NPAD, CH), jnp.float32),
        pltpu.SemaphoreType.DMA,
        pltpu.SemaphoreType.DMA,
        pltpu.SemaphoreType.DMA,
    ],
    compiler_params=_SC_PARAMS,
)
def _hops(z_hbm, hist_hbm, srcr_hbm, dstr_hbm, out_hbm,
          src_v, dst_v, rows_v, ta, th0, th1, tsv, tiv, acc_sh, g_sh,
          semi, semg, sems):
    core = lax.axis_index("c")
    sid = lax.axis_index("s")
    base = sid * RPT
    pltpu.async_copy(srcr_hbm.at[sid], src_v, semi)
    pltpu.async_copy(dstr_hbm.at[sid], dst_v, semi)

    nfull = RPT // CHUNK
    rem = RPT - nfull * CHUNK
    zeros16i = jnp.zeros((16,), jnp.int32)
    iota16 = lax.iota(jnp.int32, 16)

    for k in range(nfull + 1):
        nb = CHUNK if k < nfull else rem
        blk = pl.ds(base + k * CHUNK, nb)
        tab = ta.at[pl.ds(0, nb)]
        pltpu.sync_copy(z_hbm.at[core].at[blk], tab)
        pltpu.sync_copy(hist_hbm.at[0].at[blk], th0.at[pl.ds(0, nb)])
        pltpu.sync_copy(hist_hbm.at[1].at[blk], th1.at[pl.ds(0, nb)])

        ngf = nb // 16
        ntail = nb - ngf * 16

        def _group(g16, mask):
            ridx = g16 * 16 + iota16
            cnt0 = plsc.load_gather(th0, [ridx, zeros16i], mask=mask)
            cnt1 = plsc.load_gather(th1, [ridx, zeros16i], mask=mask)
            deg = cnt0 + cnt1 + 1.0
            inv = 1.0 / deg
            ibits = plsc.bitcast(deg, jnp.int32)
            y = plsc.bitcast(0x5F3759DF - lax.shift_right_logical(ibits, 1),
                             jnp.float32)
            for _ in range(3):
                y = y * (1.5 - 0.5 * deg * y * y)
            sk = k * CHUNK + g16 * 16
            plsc.store_scatter(tsv, [sk + iota16], y, mask=mask)
            plsc.store_scatter(tiv, [sk + iota16], inv, mask=mask)

        @pl.loop(0, ngf)
        def _(g16):
            _group(g16, iota16 >= 0)

        if ntail:
            _group(ngf, iota16 < ntail)

        @pl.loop(0, nb)
        def _(r):
            sv = _splat(tsv, k * CHUNK + r)
            for q in range(CH // 16):
                sl = pl.ds(q * 16, 16)
                ta[r, sl] = ta[r, sl] * sv

        pltpu.sync_copy(tab, g_sh.at[blk])
        pltpu.sync_copy(tab, acc_sh.at[blk])

    pltpu.make_async_copy(srcr_hbm.at[sid], src_v, semi).wait()
    pltpu.make_async_copy(dstr_hbm.at[sid], dst_v, semi).wait()
    plsc.subcore_barrier()

    _ring(g_sh, acc_sh, src_v, dst_v, rows_v, semg, sems)

    for k in range(nfull + 1):
        nb = CHUNK if k < nfull else rem
        blk = pl.ds(base + k * CHUNK, nb)
        tab = ta.at[pl.ds(0, nb)]
        pltpu.sync_copy(acc_sh.at[blk], tab)

        @pl.loop(0, nb)
        def _(r):
            iv = _splat(tiv, k * CHUNK + r)
            for q in range(CH // 16):
                sl = pl.ds(q * 16, 16)
                ta[r, sl] = ta[r, sl] * iv

        pltpu.sync_copy(tab, g_sh.at[blk])
        pltpu.sync_copy(tab, acc_sh.at[blk])

    plsc.subcore_barrier()

    _ring(g_sh, acc_sh, src_v, dst_v, rows_v, semg, sems)

    rows = pl.ds(base, RPT)
    pltpu.sync_copy(acc_sh.at[rows], out_hbm.at[core].at[rows])


def _deg_cols(hist_ref):
    cnt = hist_ref[0, :, 0:1] + hist_ref[1, :, 0:1]
    return cnt + 1.0


def _mm_body(x_ref, wt_ref, z_ref):
    z = lax.dot_general(
        x_ref[...], wt_ref[...], (((1,), (0,)), ((), ())),
        precision=lax.Precision.HIGHEST,
        preferred_element_type=jnp.float32)
    z_ref[0] = z[:, :CH]
    z_ref[1] = z[:, CH:]


def _final_body(acc_ref, hist_ref, b_ref, out_ref):
    h = jnp.concatenate([acc_ref[0], acc_ref[1]], axis=1)
    logits = lax.rsqrt(_deg_cols(hist_ref)) * h + b_ref[...]
    m = jnp.max(logits, axis=1, keepdims=True)
    shifted = logits - m
    lse = jnp.log(jnp.sum(jnp.exp(shifted), axis=1, keepdims=True))
    out_ref[...] = shifted - lse


def kernel(x, edge_index, W, b):
    src = edge_index[0]
    dst = edge_index[1]
    pad = EPAD - E
    srcp = jnp.concatenate([src, jnp.zeros((pad,), jnp.int32)])
    dstp = jnp.concatenate([dst, jnp.full((pad,), N, jnp.int32)])
    dstr32 = dstp.reshape(NW, NCH, CHUNK)
    srcr16 = srcp.reshape(NS, NCH2, CHUNK)
    dstr16 = dstp.reshape(NS, NCH2, CHUNK)

    xpad = jnp.concatenate([x, jnp.zeros((NPAD - N, D), jnp.float32)])
    RB = NPAD // 8
    zc = pl.pallas_call(
        _mm_body,
        grid=(8,),
        in_specs=[
            pl.BlockSpec((RB, D), lambda i: (i, 0)),
            pl.BlockSpec((D, C), lambda i: (0, 0)),
        ],
        out_specs=pl.BlockSpec((NC, RB, CH), lambda i: (0, i, 0)),
        out_shape=jax.ShapeDtypeStruct((NC, NPAD, CH), jnp.float32),
    )(xpad, W.T)

    hist = _hist(dstr32)

    acc2 = _hops(zc, hist, srcr16, dstr16)

    FB = N // 10
    out = pl.pallas_call(
        _final_body,
        grid=(10,),
        in_specs=[
            pl.BlockSpec((NC, FB, CH), lambda i: (0, i, 0)),
            pl.BlockSpec((NC, FB, 16), lambda i: (0, i, 0)),
            pl.BlockSpec((1, C), lambda i: (0, 0)),
        ],
        out_specs=pl.BlockSpec((FB, C), lambda i: (i, 0)),
        out_shape=jax.ShapeDtypeStruct((N, C), jnp.float32),
    )(acc2, hist, b.reshape(1, C))

    return out
